# Initial kernel scaffold; baseline (speedup 1.0000x reference)
import jax
import jax.numpy as jnp
from jax.experimental import pallas as pl


def kernel(x, edge_index, batch, edge_attr, W1, b1, W2, b2):
    raise NotImplementedError("write your pallas kernel here")


# speedup vs baseline: 3.7434x; 3.7434x over previous
import dataclasses
import functools

import jax
import jax.numpy as jnp
from jax import lax
from jax.experimental import pallas as pl
from jax.experimental.pallas import tpu as pltpu
from jax.experimental.pallas import tpu_sc as plsc

N = 10000
E = 320000
D = 128
DE = 16
H = 16
G = 64

NC = 2
NS = 16
L = 16
NW = NC * NS
PER_W = E // NW
GW = 128

def _sc_gather_body(y_hbm, src2_hbm, dst_hbm, batch_hbm, ys_hbm, gb_hbm,
                    batch_v, dst_v, gb_v):
    wid = lax.axis_index("s") * NC + lax.axis_index("c")
    base = wid * PER_W
    pltpu.sync_copy(batch_hbm, batch_v)
    pltpu.sync_copy(dst_hbm.at[pl.ds(base, PER_W)], dst_v)

    @pl.loop(0, PER_W, step=L)
    def _(i):
        d16 = dst_v[pl.ds(i, L)]
        gb_v[pl.ds(i, L)] = plsc.load_gather(batch_v, [d16])

    pltpu.sync_copy(gb_v, gb_hbm.at[pl.ds(base, PER_W)])

    def body(i_vmem, o_vmem):
        pltpu.sync_copy(y_hbm.at[i_vmem.at[0]], o_vmem)

    pltpu.emit_pipeline(
        body,
        grid=(E // GW,),
        in_specs=[pl.BlockSpec((1, GW), lambda i: (0, i))],
        out_specs=[pl.BlockSpec((GW, H), lambda i: (i, 0))],
        core_axis_name=("c", "s"),
        dimension_semantics=(pltpu.PARALLEL,),
    )(src2_hbm, ys_hbm)


@functools.cache
def _sc_gather():
    mesh = plsc.VectorSubcoreMesh(core_axis_name="c", subcore_axis_name="s")
    cp = pltpu.CompilerParams()
    if "needs_layout_passes" in pltpu.CompilerParams.__dataclass_fields__:
        cp = dataclasses.replace(cp, needs_layout_passes=False)
    if "use_tc_tiling_on_sc" in pltpu.CompilerParams.__dataclass_fields__:
        cp = dataclasses.replace(cp, use_tc_tiling_on_sc=False)
    return pl.kernel(
        _sc_gather_body,
        out_type=(
            jax.ShapeDtypeStruct((E, H), jnp.float32),
            jax.ShapeDtypeStruct((E,), jnp.int32),
        ),
        mesh=mesh,
        scratch_types=[
            pltpu.VMEM((N,), jnp.int32),
            pltpu.VMEM((PER_W,), jnp.int32),
            pltpu.VMEM((PER_W,), jnp.int32),
        ],
        compiler_params=cp,
    )


def _y_body(x_ref, w_ref, y_ref):
    y_ref[...] = jnp.dot(x_ref[...], w_ref[...],
                         preferred_element_type=jnp.float32)


def _msg_body(ys_ref, ea_ref, gb_ref, w1e_ref, b1_ref, w2_ref, b2_ref,
              acc_ref, *, eb):
    a = jnp.dot(ea_ref[...], w1e_ref[...], preferred_element_type=jnp.float32)
    h = jnp.maximum(ys_ref[...] + a + b1_ref[...], 0.0)
    msg = jnp.dot(h, w2_ref[...], preferred_element_type=jnp.float32)
    msg = msg + b2_ref[0, 0]
    gb = gb_ref[0, 0, :]
    oh = gb[:, None] == lax.broadcasted_iota(jnp.int32, (eb, G), 1)
    contrib = jnp.sum(jnp.where(oh, msg, 0.0), axis=0)

    @pl.when(pl.program_id(0) == 0)
    def _():
        acc_ref[...] = jnp.zeros_like(acc_ref)

    acc_ref[...] += contrib[None, :]


def kernel(x, edge_index, batch, edge_attr, W1, b1, W2, b2):
    src = edge_index[0]
    dst = edge_index[1]
    w1x = W1[:D]
    w1e = W1[D:]
    w2h = (W2 * 0.5).astype(jnp.float32)
    b2h = (b2 * 0.5).reshape(1, 1).astype(jnp.float32)
    b1r = b1.reshape(1, H)

    nb = 10
    y = pl.pallas_call(
        _y_body,
        grid=(nb,),
        in_specs=[
            pl.BlockSpec((N // nb, D), lambda i: (i, 0)),
            pl.BlockSpec((D, H), lambda i: (0, 0)),
        ],
        out_specs=pl.BlockSpec((N // nb, H), lambda i: (i, 0)),
        out_shape=jax.ShapeDtypeStruct((N, H), jnp.float32),
    )(x, w1x)

    ys, gb = _sc_gather()(y, src.reshape(1, E), dst, batch)

    eb = 8000
    nbe = E // eb
    gb3 = gb.reshape(nbe, 1, eb)
    acc = pl.pallas_call(
        functools.partial(_msg_body, eb=eb),
        grid=(nbe,),
        in_specs=[
            pl.BlockSpec((eb, H), lambda i: (i, 0)),
            pl.BlockSpec((eb, DE), lambda i: (i, 0)),
            pl.BlockSpec((1, 1, eb), lambda i: (i, 0, 0)),
            pl.BlockSpec((DE, H), lambda i: (0, 0)),
            pl.BlockSpec((1, H), lambda i: (0, 0)),
            pl.BlockSpec((H, 1), lambda i: (0, 0)),
            pl.BlockSpec((1, 1), lambda i: (0, 0)),
        ],
        out_specs=pl.BlockSpec((1, G), lambda i: (0, 0)),
        out_shape=jax.ShapeDtypeStruct((1, G), jnp.float32),
    )(ys, edge_attr, gb3, w1e, b1r, w2h, b2h)

    return acc[0]


# speedup vs baseline: 8.1672x; 2.1818x over previous
import dataclasses
import functools

import jax
import jax.numpy as jnp
from jax import lax
from jax.experimental import pallas as pl
from jax.experimental.pallas import tpu as pltpu
from jax.experimental.pallas import tpu_sc as plsc

N = 10000
E = 320000
D = 128
DE = 16
H = 16
G = 64

NC = 2
NS = 16
L = 16
NW = NC * NS
CH = 80
KCH = 5
SUP = CH * KCH


def _make_sc_body(edge0, per_w, nsup):
    def body(y_hbm, src_hbm, dst_hbm, batch_hbm, eat_hbm,
             ys_hbm, gb_hbm, ea_hbm,
             src_v, dst_v, batch_v, gb_v, p0, p1, p2, p3,
             e0, e1, t0, t1, gg, ss, ge, se):
        pbuf = (p0, p1, p2, p3)
        ebuf = (e0, e1)
        stg = (t0, t1)
        wid = lax.axis_index("s") * NC + lax.axis_index("c")
        obase = wid * per_w
        ibase = edge0 + obase

        pltpu.sync_copy(src_hbm.at[pl.ds(ibase, per_w)], src_v)

        def fire(s, p):
            @pl.loop(0, KCH)
            def _(k):
                off = s * SUP + k * CH
                pltpu.async_copy(y_hbm.at[src_v.at[pl.ds(off, CH)]],
                                 p.at[pl.ds(k * CH, CH)], gg)

        def drain_gathers(p):
            pltpu.make_async_copy(ys_hbm.at[pl.ds(obase, SUP)], p, gg).wait()

        def load_ea(s, eb):
            for k in range(DE):
                pltpu.async_copy(
                    eat_hbm.at[k, pl.ds(ibase + s * SUP, SUP)],
                    eb.at[pl.ds(k * SUP, SUP)], ge)

        def drain_ea():
            pltpu.make_async_copy(eat_hbm.at[0, pl.ds(0, DE * SUP)],
                                  ebuf[0], ge).wait()

        fire(0, pbuf[0])
        pltpu.sync_copy(batch_hbm, batch_v)
        pltpu.sync_copy(dst_hbm.at[pl.ds(ibase, per_w)], dst_v)
        fire(1, pbuf[1])
        load_ea(0, ebuf[0])

        @pl.loop(0, per_w, step=L)
        def _(i):
            gb_v[pl.ds(i, L)] = plsc.load_gather(batch_v,
                                                 [dst_v[pl.ds(i, L)]])

        pltpu.sync_copy(gb_v, gb_hbm.at[pl.ds(obase, per_w)])

        iot = lax.iota(jnp.int32, L)
        for s in range(nsup):
            drain_gathers(pbuf[s % 4])
            pltpu.async_copy(pbuf[s % 4],
                             ys_hbm.at[pl.ds(obase + s * SUP, SUP)], ss)
            if s >= 2:
                pltpu.make_async_copy(
                    pbuf[(s - 2) % 4],
                    ys_hbm.at[pl.ds(obase + (s - 2) * SUP, SUP)], ss).wait()
            if s + 2 < nsup:
                fire(s + 2, pbuf[(s + 2) % 4])
            if s + 1 < nsup:
                load_ea(s + 1, ebuf[(s + 1) % 2])
            drain_ea()
            eb = ebuf[s % 2]
            st = stg[s % 2]
            if s >= 2:
                pltpu.make_async_copy(
                    stg[s % 2], ea_hbm.at[pl.ds((obase + (s - 2) * SUP) * DE,
                                                SUP * DE)], se).wait()

            @pl.loop(0, SUP, step=L)
            def _(e):
                for k in range(DE):
                    v = eb[pl.ds(k * SUP + e, L)]
                    plsc.store_scatter(st, [(e + iot) * DE + k], v)

            pltpu.async_copy(st, ea_hbm.at[pl.ds((obase + s * SUP) * DE,
                                                 SUP * DE)], se)
        for s in (nsup - 2, nsup - 1):
            pltpu.make_async_copy(
                pbuf[s % 4], ys_hbm.at[pl.ds(obase + s * SUP, SUP)],
                ss).wait()
            pltpu.make_async_copy(
                stg[s % 2], ea_hbm.at[pl.ds((obase + s * SUP) * DE,
                                            SUP * DE)], se).wait()

    return body


@functools.cache
def _sc_gather(edge0, e_cnt):
    mesh = plsc.VectorSubcoreMesh(core_axis_name="c", subcore_axis_name="s")
    cp = pltpu.CompilerParams()
    if "needs_layout_passes" in pltpu.CompilerParams.__dataclass_fields__:
        cp = dataclasses.replace(cp, needs_layout_passes=False)
    if "use_tc_tiling_on_sc" in pltpu.CompilerParams.__dataclass_fields__:
        cp = dataclasses.replace(cp, use_tc_tiling_on_sc=False)
    per_w = e_cnt // NW
    nsup = per_w // SUP
    return pl.kernel(
        _make_sc_body(edge0, per_w, nsup),
        out_type=(
            jax.ShapeDtypeStruct((e_cnt, H), jnp.float32),
            jax.ShapeDtypeStruct((e_cnt,), jnp.int32),
            jax.ShapeDtypeStruct((e_cnt * DE,), jnp.float32),
        ),
        mesh=mesh,
        scratch_types=[
            pltpu.VMEM((per_w,), jnp.int32),
            pltpu.VMEM((per_w,), jnp.int32),
            pltpu.VMEM((N,), jnp.int32),
            pltpu.VMEM((per_w,), jnp.int32),
            pltpu.VMEM((SUP, H), jnp.float32),
            pltpu.VMEM((SUP, H), jnp.float32),
            pltpu.VMEM((SUP, H), jnp.float32),
            pltpu.VMEM((SUP, H), jnp.float32),
            pltpu.VMEM((DE * SUP,), jnp.float32),
            pltpu.VMEM((DE * SUP,), jnp.float32),
            pltpu.VMEM((SUP * DE,), jnp.float32),
            pltpu.VMEM((SUP * DE,), jnp.float32),
            pltpu.SemaphoreType.DMA,
            pltpu.SemaphoreType.DMA,
            pltpu.SemaphoreType.DMA,
            pltpu.SemaphoreType.DMA,
        ],
        compiler_params=cp,
    )


def _y_body(x_ref, w_ref, y_ref):
    y_ref[...] = jnp.dot(x_ref[...], w_ref[...],
                         preferred_element_type=jnp.float32)


def _msg_body(ys_ref, ea_ref, gb_ref, w1e_ref, b1_ref, w2_ref, b2_ref,
              perm_ref, acc_ref, *, rows, nblk, last_valid):
    m = rows // 16
    a = jnp.dot(ea_ref[...], w1e_ref[...], preferred_element_type=jnp.float32)
    h = jnp.maximum(ys_ref[...] + a + b1_ref[...], 0.0)
    msg8 = jnp.dot(h, w2_ref[...], preferred_element_type=jnp.float32)
    msg8 = msg8 + b2_ref[0, 0]
    valid = jnp.where(pl.program_id(0) == nblk - 1, last_valid, m)
    vmask8 = lax.broadcasted_iota(jnp.int32, (rows, 8), 0) < 16 * valid
    msg8 = jnp.where(vmask8, msg8, 0.0)
    p = jnp.dot(perm_ref[...], msg8, preferred_element_type=jnp.float32)
    msgw = jnp.concatenate([p[m * t:m * (t + 1), :] for t in range(16)],
                           axis=1)
    gbw = gb_ref[...]
    cols = [
        jnp.sum(jnp.where(gbw == g, msgw, 0.0), axis=0)
        for g in range(G)
    ]
    contrib = jnp.sum(jnp.stack(cols), axis=1)

    @pl.when(pl.program_id(0) == 0)
    def _():
        acc_ref[...] = jnp.zeros_like(acc_ref)

    acc_ref[...] += contrib[None, :]


def _blockdiag8(w):
    eye8 = jnp.eye(8, dtype=w.dtype)
    a, b = w.shape
    return (eye8[:, None, :, None] * w[None, :, None, :]).reshape(8 * a, 8 * b)


def kernel(x, edge_index, batch, edge_attr, W1, b1, W2, b2):
    src = edge_index[0]
    dst = edge_index[1]
    w1x = W1[:D]
    w1e = W1[D:]
    bw1x = _blockdiag8(w1x)
    bw1e = _blockdiag8(w1e)
    bw2 = _blockdiag8(W2 * 0.5)
    b1t = jnp.tile(b1, 8).reshape(1, 128)
    b2h = (b2 * 0.5).reshape(1, 1).astype(jnp.float32)

    x3 = x.reshape(N // 8, 8 * D)
    y128 = pl.pallas_call(
        _y_body,
        out_shape=jax.ShapeDtypeStruct((N // 8, 128), jnp.float32),
    )(x3, bw1x)
    y = y128.reshape(N, H)

    eb = 8192
    rows = eb // 8
    m = eb // 128
    ii = lax.broadcasted_iota(jnp.int32, (rows, rows), 0)
    jj = lax.broadcasted_iota(jnp.int32, (rows, rows), 1)
    perm = (jj == 16 * (ii % m) + ii // m).astype(jnp.float32)
    eat = jnp.transpose(edge_attr)

    def msg_half(ys, gb, ea_lin, e_cnt):
        nblk = -(-e_cnt // eb)
        last_valid = (e_cnt - (nblk - 1) * eb) // 128
        return pl.pallas_call(
            functools.partial(_msg_body, rows=rows, nblk=nblk,
                              last_valid=last_valid),
            grid=(nblk,),
            in_specs=[
                pl.BlockSpec((rows, 128), lambda i: (i, 0)),
                pl.BlockSpec((rows, 128), lambda i: (i, 0)),
                pl.BlockSpec((m, 128), lambda i: (i, 0)),
                pl.BlockSpec((128, 128), lambda i: (0, 0)),
                pl.BlockSpec((1, 128), lambda i: (0, 0)),
                pl.BlockSpec((128, 8), lambda i: (0, 0)),
                pl.BlockSpec((1, 1), lambda i: (0, 0)),
                pl.BlockSpec((rows, rows), lambda i: (0, 0)),
            ],
            out_specs=pl.BlockSpec((1, G), lambda i: (0, 0)),
            out_shape=jax.ShapeDtypeStruct((1, G), jnp.float32),
        )(ys.reshape(e_cnt // 8, 128), ea_lin.reshape(e_cnt // 8, 128),
          gb.reshape(e_cnt // 128, 128), bw1e, b1t, bw2, b2h, perm)

    splits = ((0, 128000), (128000, 192000))
    parts = [_sc_gather(e0, cnt)(y, src, dst, batch, eat)
             for e0, cnt in splits]
    acc = None
    for (ys, gb, ea_lin), (_, cnt) in zip(parts, splits):
        a = msg_half(ys, gb, ea_lin, cnt)
        acc = a if acc is None else acc + a

    return acc[0]
